# baseline (speedup 1.0000x reference)
import functools

import jax
import jax.numpy as jnp
from jax import lax
from jax.experimental import pallas as pl
from jax.experimental.pallas import tpu as pltpu
from jax.experimental.pallas import tpu_sc as plsc

N = 10000
E = 320000
F = 128
K = 3
G = 64
C = 40

NC = 2
NS = 16
NW = NC * NS
FPW = F // NW
EPW = E // NW
CH = 2000
L = 16

_mesh = plsc.VectorSubcoreMesh(core_axis_name="c", subcore_axis_name="s")


def _wid():
    return lax.axis_index("s") * NC + lax.axis_index("c")


@functools.partial(
    pl.kernel,
    out_type=jax.ShapeDtypeStruct((NW, N), jnp.float32),
    mesh=_mesh,
    compiler_params=pltpu.CompilerParams(needs_layout_passes=False),
    scratch_types=[
        pltpu.VMEM((N,), jnp.float32),
        pltpu.VMEM((CH,), jnp.int32),
        pltpu.VMEM((CH,), jnp.float32),
    ],
)
def _deg_kernel(dst_hbm, ew_hbm, out_hbm, acc_v, dst_v, ew_v):
    w = _wid()

    def zero_body(i, _):
        acc_v[pl.ds(i * L, L)] = jnp.zeros((L,), jnp.float32)
        return 0

    lax.fori_loop(0, N // L, zero_body, 0)

    base = w * EPW

    def chunk_body(ch, _):
        off = base + ch * CH
        pltpu.sync_copy(dst_hbm.at[pl.ds(off, CH)], dst_v)
        pltpu.sync_copy(ew_hbm.at[pl.ds(off, CH)], ew_v)

        def blk_body(j, _):
            d16 = dst_v[pl.ds(j * L, L)]
            w16 = ew_v[pl.ds(j * L, L)]
            plsc.addupdate_scatter(acc_v, [d16], w16)
            return 0

        lax.fori_loop(0, CH // L, blk_body, 0)
        return 0

    lax.fori_loop(0, EPW // CH, chunk_body, 0)
    pltpu.sync_copy(acc_v, out_hbm.at[w])


@functools.partial(
    pl.kernel,
    out_type=jax.ShapeDtypeStruct((E,), jnp.float32),
    mesh=_mesh,
    compiler_params=pltpu.CompilerParams(needs_layout_passes=False),
    scratch_types=[
        pltpu.VMEM((N,), jnp.float32),
        pltpu.VMEM((CH,), jnp.int32),
        pltpu.VMEM((CH,), jnp.int32),
        pltpu.VMEM((CH,), jnp.float32),
        pltpu.VMEM((CH,), jnp.float32),
    ],
)
def _norm_kernel(src_hbm, dst_hbm, ew_hbm, dinv_hbm, out_hbm,
                 dinv_v, src_v, dst_v, ew_v, nrm_v):
    w = _wid()
    pltpu.sync_copy(dinv_hbm, dinv_v)
    base = w * EPW

    def chunk_body(ch, _):
        off = base + ch * CH
        pltpu.sync_copy(src_hbm.at[pl.ds(off, CH)], src_v)
        pltpu.sync_copy(dst_hbm.at[pl.ds(off, CH)], dst_v)
        pltpu.sync_copy(ew_hbm.at[pl.ds(off, CH)], ew_v)

        def blk_body(j, _):
            s16 = src_v[pl.ds(j * L, L)]
            d16 = dst_v[pl.ds(j * L, L)]
            w16 = ew_v[pl.ds(j * L, L)]
            a = plsc.load_gather(dinv_v, [s16])
            b = plsc.load_gather(dinv_v, [d16])
            nrm_v[pl.ds(j * L, L)] = a * w16 * b
            return 0

        lax.fori_loop(0, CH // L, blk_body, 0)
        pltpu.sync_copy(nrm_v, out_hbm.at[pl.ds(off, CH)])
        return 0

    lax.fori_loop(0, EPW // CH, chunk_body, 0)


NP = 10240
BLK = 64
BPS = 160
EPS = BLK * BPS
EP = EPS * NW
IDXC = 8
NBR = NP // NS
ZBR = 64
NPAIR = BPS // 2


@functools.partial(
    pl.kernel,
    out_type=jax.ShapeDtypeStruct((NC, NP, F), jnp.float32),
    mesh=_mesh,
    compiler_params=pltpu.CompilerParams(needs_layout_passes=False),
    scratch_types=[
        pltpu.VMEM_SHARED((NP, F), jnp.float32),
        pltpu.VMEM((BLK, F), jnp.float32),
        pltpu.VMEM((BLK,), jnp.int32),
        pltpu.VMEM((BLK,), jnp.int32),
        pltpu.VMEM((BLK,), jnp.float32),
        pltpu.VMEM((ZBR, F), jnp.float32),
        pltpu.SemaphoreType.DMA,
    ],
)
def _step_kernel(tab_hbm, src_hbm, dst_hbm, nrm_hbm, out_hbm,
                 acc_sh, rows_v, src_v, dst_v, nrm_v, zb_v, sem):
    c = lax.axis_index("c")
    s = lax.axis_index("s")
    row0 = s * NBR

    @plsc.parallel_loop(0, ZBR, unroll=8)
    def zb_body(r):
        for u in range(F // L):
            zb_v[r, pl.ds(u * L, L)] = jnp.zeros((L,), jnp.float32)

    for t in range(NBR // ZBR):
        pltpu.sync_copy(zb_v, acc_sh.at[pl.ds(row0 + t * ZBR, ZBR)])
    plsc.subcore_barrier()

    ebase = (c * NS + s) * EPS

    def chunk_body(ch, _):
        off = ebase + ch * BLK
        pltpu.sync_copy(src_hbm.at[pl.ds(off, BLK)], src_v)
        pltpu.sync_copy(dst_hbm.at[pl.ds(off, BLK)], dst_v)
        pltpu.sync_copy(nrm_hbm.at[pl.ds(off, BLK)], nrm_v)
        pltpu.async_copy(tab_hbm.at[src_v], rows_v, sem).wait()

        @plsc.parallel_loop(0, BLK // L, unroll=1)
        def scale_body(j):
            n16 = nrm_v[pl.ds(j * L, L)]
            for l in range(L):
                e = j * L + l
                n = n16[l]
                for u in range(F // L):
                    fs = pl.ds(u * L, L)
                    rows_v[e, fs] = rows_v[e, fs] * n

        pltpu.sync_copy(rows_v, acc_sh.at[dst_v], add=True)
        return 0

    lax.fori_loop(0, BPS, chunk_body, 0)
    plsc.subcore_barrier()
    band = pl.ds(row0, NBR)
    pltpu.sync_copy(acc_sh.at[band], out_hbm.at[c].at[band])


def _bn(h, gamma, beta, eps=1e-5):
    mu = h.mean(axis=0)
    var = h.var(axis=0)
    return (h - mu) / jnp.sqrt(var + eps) * gamma + beta


def kernel(x, edge_index, batch, edge_attr, W1, b1, g1, be1, W2, b2, g2, be2,
           Wm, bm, gm, bem, Wf1, bf1, gf1, bef1, Wf2, bf2, gf2, bef2, Wf3, bf3):
    src = edge_index[0]
    dst = edge_index[1]

    deg_parts = _deg_kernel(dst, edge_attr)
    deg = deg_parts.sum(axis=0)
    dinv = jnp.where(deg > 0, jax.lax.rsqrt(jnp.maximum(deg, 1e-30)), 0.0)
    nrm = _norm_kernel(src, dst, edge_attr, dinv)

    src_p = jnp.pad(src, (0, EP - E))
    dst_p = jnp.pad(dst, (0, EP - E))
    nrm_p = jnp.pad(nrm, (0, EP - E))

    def conv(h_pad):
        hs = []
        t = h_pad
        for _ in range(K):
            parts = _step_kernel(t, src_p, dst_p, nrm_p)
            t = parts[0] + parts[1]
            hs.append(t[:N])
        return hs

    x_pad = jnp.pad(x, ((0, NP - N), (0, 0)))
    h1, h2, h3 = conv(x_pad)
    hcat = jnp.concatenate([x, h1, h2, h3], axis=1)
    x1 = _bn(jax.nn.relu(hcat @ W1 + b1), g1, be1)

    h1, h2, h3 = conv(jnp.pad(x1, ((0, NP - N), (0, 0))))
    hcat = jnp.concatenate([x1, h1, h2, h3], axis=1)
    x2 = _bn(jax.nn.relu(hcat @ W2 + b2), g2, be2)

    out = jnp.concatenate([x1, x2], axis=1)
    out = _bn(jax.nn.relu(out @ Wm + bm), gm, bem)
    out = jax.ops.segment_max(out, batch, num_segments=G)
    out = _bn(jax.nn.relu(out @ Wf1 + bf1), gf1, bef1)
    out = _bn(jax.nn.relu(out @ Wf2 + bf2), gf2, bef2)
    out = out @ Wf3 + bf3
    return jax.nn.log_softmax(out, axis=-1)


# speedup vs baseline: 2.2635x; 2.2635x over previous
import functools

import jax
import jax.numpy as jnp
from jax import lax
from jax.experimental import pallas as pl
from jax.experimental.pallas import tpu as pltpu
from jax.experimental.pallas import tpu_sc as plsc

N = 10000
E = 320000
F = 128
K = 3
G = 64
C = 40

NC = 2
NS = 16
NW = NC * NS
FPW = F // NW
EPW = E // NW
CH = 2000
L = 16

_mesh = plsc.VectorSubcoreMesh(core_axis_name="c", subcore_axis_name="s")


def _wid():
    return lax.axis_index("s") * NC + lax.axis_index("c")


@functools.partial(
    pl.kernel,
    out_type=jax.ShapeDtypeStruct((NW, N), jnp.float32),
    mesh=_mesh,
    compiler_params=pltpu.CompilerParams(needs_layout_passes=False),
    scratch_types=[
        pltpu.VMEM((N,), jnp.float32),
        pltpu.VMEM((CH,), jnp.int32),
        pltpu.VMEM((CH,), jnp.float32),
    ],
)
def _deg_kernel(dst_hbm, ew_hbm, out_hbm, acc_v, dst_v, ew_v):
    w = _wid()

    def zero_body(i, _):
        acc_v[pl.ds(i * L, L)] = jnp.zeros((L,), jnp.float32)
        return 0

    lax.fori_loop(0, N // L, zero_body, 0)

    base = w * EPW

    def chunk_body(ch, _):
        off = base + ch * CH
        pltpu.sync_copy(dst_hbm.at[pl.ds(off, CH)], dst_v)
        pltpu.sync_copy(ew_hbm.at[pl.ds(off, CH)], ew_v)

        def blk_body(j, _):
            d16 = dst_v[pl.ds(j * L, L)]
            w16 = ew_v[pl.ds(j * L, L)]
            plsc.addupdate_scatter(acc_v, [d16], w16)
            return 0

        lax.fori_loop(0, CH // L, blk_body, 0)
        return 0

    lax.fori_loop(0, EPW // CH, chunk_body, 0)
    pltpu.sync_copy(acc_v, out_hbm.at[w])


@functools.partial(
    pl.kernel,
    out_type=jax.ShapeDtypeStruct((E,), jnp.float32),
    mesh=_mesh,
    compiler_params=pltpu.CompilerParams(needs_layout_passes=False),
    scratch_types=[
        pltpu.VMEM((N,), jnp.float32),
        pltpu.VMEM((CH,), jnp.int32),
        pltpu.VMEM((CH,), jnp.int32),
        pltpu.VMEM((CH,), jnp.float32),
        pltpu.VMEM((CH,), jnp.float32),
    ],
)
def _norm_kernel(src_hbm, dst_hbm, ew_hbm, dinv_hbm, out_hbm,
                 dinv_v, src_v, dst_v, ew_v, nrm_v):
    w = _wid()
    pltpu.sync_copy(dinv_hbm, dinv_v)
    base = w * EPW

    def chunk_body(ch, _):
        off = base + ch * CH
        pltpu.sync_copy(src_hbm.at[pl.ds(off, CH)], src_v)
        pltpu.sync_copy(dst_hbm.at[pl.ds(off, CH)], dst_v)
        pltpu.sync_copy(ew_hbm.at[pl.ds(off, CH)], ew_v)

        def blk_body(j, _):
            s16 = src_v[pl.ds(j * L, L)]
            d16 = dst_v[pl.ds(j * L, L)]
            w16 = ew_v[pl.ds(j * L, L)]
            a = plsc.load_gather(dinv_v, [s16])
            b = plsc.load_gather(dinv_v, [d16])
            nrm_v[pl.ds(j * L, L)] = a * w16 * b
            return 0

        lax.fori_loop(0, CH // L, blk_body, 0)
        pltpu.sync_copy(nrm_v, out_hbm.at[pl.ds(off, CH)])
        return 0

    lax.fori_loop(0, EPW // CH, chunk_body, 0)


NP = 10240
BLK = 128
BPS = 80
EPS = BLK * BPS
EP = EPS * NW
IDXC = 8
NBR = NP // NS
ZBR = 64
NPAIR = BPS // 2


@functools.partial(
    pl.kernel,
    out_type=jax.ShapeDtypeStruct((NC, NP, F), jnp.float32),
    mesh=_mesh,
    compiler_params=pltpu.CompilerParams(needs_layout_passes=False),
    scratch_types=[
        pltpu.VMEM_SHARED((NP, F), jnp.float32),
        pltpu.VMEM((BLK, F), jnp.float32),
        pltpu.VMEM((BLK,), jnp.int32),
        pltpu.VMEM((BLK,), jnp.int32),
        pltpu.VMEM((BLK,), jnp.float32),
        pltpu.VMEM((ZBR, F), jnp.float32),
        pltpu.SemaphoreType.DMA,
    ],
)
def _step_kernel(tab_hbm, src_hbm, dst_hbm, nrm_hbm, out_hbm,
                 acc_sh, rows_v, src_v, dst_v, nrm_v, zb_v, sem):
    c = lax.axis_index("c")
    s = lax.axis_index("s")
    row0 = s * NBR

    @plsc.parallel_loop(0, ZBR, unroll=8)
    def zb_body(r):
        for u in range(F // L):
            zb_v[r, pl.ds(u * L, L)] = jnp.zeros((L,), jnp.float32)

    for t in range(NBR // ZBR):
        pltpu.sync_copy(zb_v, acc_sh.at[pl.ds(row0 + t * ZBR, ZBR)])
    plsc.subcore_barrier()

    ebase = (c * NS + s) * EPS

    def chunk_body(ch, _):
        off = ebase + ch * BLK
        pltpu.sync_copy(src_hbm.at[pl.ds(off, BLK)], src_v)
        pltpu.sync_copy(dst_hbm.at[pl.ds(off, BLK)], dst_v)
        pltpu.sync_copy(nrm_hbm.at[pl.ds(off, BLK)], nrm_v)
        pltpu.async_copy(tab_hbm.at[src_v], rows_v, sem).wait()

        @plsc.parallel_loop(0, BLK // L, unroll=1)
        def scale_body(j):
            n16 = nrm_v[pl.ds(j * L, L)]
            for l in range(L):
                e = j * L + l
                n = n16[l]
                for u in range(F // L):
                    fs = pl.ds(u * L, L)
                    rows_v[e, fs] = rows_v[e, fs] * n

        pltpu.sync_copy(rows_v, acc_sh.at[dst_v], add=True)
        return 0

    lax.fori_loop(0, BPS, chunk_body, 0)
    plsc.subcore_barrier()
    band = pl.ds(row0, NBR)
    pltpu.sync_copy(acc_sh.at[band], out_hbm.at[c].at[band])


def _bn(h, gamma, beta, eps=1e-5):
    mu = h.mean(axis=0)
    var = h.var(axis=0)
    return (h - mu) / jnp.sqrt(var + eps) * gamma + beta


def kernel(x, edge_index, batch, edge_attr, W1, b1, g1, be1, W2, b2, g2, be2,
           Wm, bm, gm, bem, Wf1, bf1, gf1, bef1, Wf2, bf2, gf2, bef2, Wf3, bf3):
    src = edge_index[0]
    dst = edge_index[1]

    deg_parts = _deg_kernel(dst, edge_attr)
    deg = deg_parts.sum(axis=0)
    dinv = jnp.where(deg > 0, jax.lax.rsqrt(jnp.maximum(deg, 1e-30)), 0.0)
    nrm = _norm_kernel(src, dst, edge_attr, dinv)

    pad_idx = (jnp.arange(EP - E, dtype=jnp.int32) * 13) % N
    src_p = jnp.concatenate([src, pad_idx])
    dst_p = jnp.concatenate([dst, pad_idx])
    nrm_p = jnp.pad(nrm, (0, EP - E))

    def conv(h_pad):
        hs = []
        t = h_pad
        for _ in range(K):
            parts = _step_kernel(t, src_p, dst_p, nrm_p)
            t = parts[0] + parts[1]
            hs.append(t[:N])
        return hs

    x_pad = jnp.pad(x, ((0, NP - N), (0, 0)))
    h1, h2, h3 = conv(x_pad)
    hcat = jnp.concatenate([x, h1, h2, h3], axis=1)
    x1 = _bn(jax.nn.relu(hcat @ W1 + b1), g1, be1)

    h1, h2, h3 = conv(jnp.pad(x1, ((0, NP - N), (0, 0))))
    hcat = jnp.concatenate([x1, h1, h2, h3], axis=1)
    x2 = _bn(jax.nn.relu(hcat @ W2 + b2), g2, be2)

    out = jnp.concatenate([x1, x2], axis=1)
    out = _bn(jax.nn.relu(out @ Wm + bm), gm, bem)
    out = jax.ops.segment_max(out, batch, num_segments=G)
    out = _bn(jax.nn.relu(out @ Wf1 + bf1), gf1, bef1)
    out = _bn(jax.nn.relu(out @ Wf2 + bf2), gf2, bef2)
    out = out @ Wf3 + bf3
    return jax.nn.log_softmax(out, axis=-1)


# speedup vs baseline: 3.0823x; 1.3617x over previous
import functools

import jax
import jax.numpy as jnp
from jax import lax
from jax.experimental import pallas as pl
from jax.experimental.pallas import tpu as pltpu
from jax.experimental.pallas import tpu_sc as plsc

N = 10000
E = 320000
F = 128
K = 3
G = 64
C = 40

NC = 2
NS = 16
NW = NC * NS
FPW = F // NW
EPW = E // NW
CH = 2000
L = 16

_mesh = plsc.VectorSubcoreMesh(core_axis_name="c", subcore_axis_name="s")


def _wid():
    return lax.axis_index("s") * NC + lax.axis_index("c")


@functools.partial(
    pl.kernel,
    out_type=jax.ShapeDtypeStruct((NW, N), jnp.float32),
    mesh=_mesh,
    compiler_params=pltpu.CompilerParams(needs_layout_passes=False),
    scratch_types=[
        pltpu.VMEM((N,), jnp.float32),
        pltpu.VMEM((CH,), jnp.int32),
        pltpu.VMEM((CH,), jnp.float32),
    ],
)
def _deg_kernel(dst_hbm, ew_hbm, out_hbm, acc_v, dst_v, ew_v):
    w = _wid()

    def zero_body(i, _):
        acc_v[pl.ds(i * L, L)] = jnp.zeros((L,), jnp.float32)
        return 0

    lax.fori_loop(0, N // L, zero_body, 0)

    base = w * EPW

    def chunk_body(ch, _):
        off = base + ch * CH
        pltpu.sync_copy(dst_hbm.at[pl.ds(off, CH)], dst_v)
        pltpu.sync_copy(ew_hbm.at[pl.ds(off, CH)], ew_v)

        def blk_body(j, _):
            d16 = dst_v[pl.ds(j * L, L)]
            w16 = ew_v[pl.ds(j * L, L)]
            plsc.addupdate_scatter(acc_v, [d16], w16)
            return 0

        lax.fori_loop(0, CH // L, blk_body, 0)
        return 0

    lax.fori_loop(0, EPW // CH, chunk_body, 0)
    pltpu.sync_copy(acc_v, out_hbm.at[w])


@functools.partial(
    pl.kernel,
    out_type=jax.ShapeDtypeStruct((E,), jnp.float32),
    mesh=_mesh,
    compiler_params=pltpu.CompilerParams(needs_layout_passes=False),
    scratch_types=[
        pltpu.VMEM((N,), jnp.float32),
        pltpu.VMEM((CH,), jnp.int32),
        pltpu.VMEM((CH,), jnp.int32),
        pltpu.VMEM((CH,), jnp.float32),
        pltpu.VMEM((CH,), jnp.float32),
    ],
)
def _norm_kernel(src_hbm, dst_hbm, ew_hbm, dinv_hbm, out_hbm,
                 dinv_v, src_v, dst_v, ew_v, nrm_v):
    w = _wid()
    pltpu.sync_copy(dinv_hbm, dinv_v)
    base = w * EPW

    def chunk_body(ch, _):
        off = base + ch * CH
        pltpu.sync_copy(src_hbm.at[pl.ds(off, CH)], src_v)
        pltpu.sync_copy(dst_hbm.at[pl.ds(off, CH)], dst_v)
        pltpu.sync_copy(ew_hbm.at[pl.ds(off, CH)], ew_v)

        def blk_body(j, _):
            s16 = src_v[pl.ds(j * L, L)]
            d16 = dst_v[pl.ds(j * L, L)]
            w16 = ew_v[pl.ds(j * L, L)]
            a = plsc.load_gather(dinv_v, [s16])
            b = plsc.load_gather(dinv_v, [d16])
            nrm_v[pl.ds(j * L, L)] = a * w16 * b
            return 0

        lax.fori_loop(0, CH // L, blk_body, 0)
        pltpu.sync_copy(nrm_v, out_hbm.at[pl.ds(off, CH)])
        return 0

    lax.fori_loop(0, EPW // CH, chunk_body, 0)


NP = 10240
BLK = 128
BPS = 80
EPS = BLK * BPS
EP = EPS * NW
IDXC = 8
NBR = NP // NS
ZBR = 64
NPAIR = BPS // 2


@functools.partial(
    pl.kernel,
    out_type=jax.ShapeDtypeStruct((NC, NP, F), jnp.float32),
    mesh=_mesh,
    compiler_params=pltpu.CompilerParams(needs_layout_passes=False),
    scratch_types=[
        pltpu.VMEM_SHARED((NP, F), jnp.float32),
        pltpu.VMEM((BLK, F), jnp.float32),
        pltpu.VMEM((BLK, F), jnp.float32),
        pltpu.VMEM((BLK,), jnp.int32),
        pltpu.VMEM((BLK,), jnp.int32),
        pltpu.VMEM((BLK,), jnp.float32),
        pltpu.VMEM((BLK,), jnp.int32),
        pltpu.VMEM((BLK,), jnp.int32),
        pltpu.VMEM((BLK,), jnp.float32),
        pltpu.VMEM((ZBR, F), jnp.float32),
        pltpu.SemaphoreType.DMA,
        pltpu.SemaphoreType.DMA,
        pltpu.SemaphoreType.DMA,
    ],
)
def _step_kernel(tab_hbm, src_hbm, dst_hbm, nrm_hbm, out_hbm,
                 acc_sh, rows_a, rows_b, src_v, dst_v, nrm_v,
                 src_w, dst_w, nrm_w, zb_v, sem_ga, sem_gb, sem_sa):
    c = lax.axis_index("c")
    s = lax.axis_index("s")
    row0 = s * NBR

    @plsc.parallel_loop(0, ZBR, unroll=8)
    def zb_body(r):
        for u in range(F // L):
            zb_v[r, pl.ds(u * L, L)] = jnp.zeros((L,), jnp.float32)

    for t in range(NBR // ZBR):
        pltpu.sync_copy(zb_v, acc_sh.at[pl.ds(row0 + t * ZBR, ZBR)])
    plsc.subcore_barrier()

    ebase = (c * NS + s) * EPS

    def idx_load(b, sv, dv, nv):
        off = ebase + b * BLK
        pltpu.sync_copy(src_hbm.at[pl.ds(off, BLK)], sv)
        pltpu.sync_copy(dst_hbm.at[pl.ds(off, BLK)], dv)
        pltpu.sync_copy(nrm_hbm.at[pl.ds(off, BLK)], nv)

    def scale(buf, nv):
        @plsc.parallel_loop(0, BLK // L, unroll=1)
        def scale_body(j):
            n16 = nv[pl.ds(j * L, L)]
            for l in range(L):
                e = j * L + l
                n = n16[l]
                for u in range(F // L):
                    fs = pl.ds(u * L, L)
                    buf[e, fs] = buf[e, fs] * n

    idx_load(0, src_v, dst_v, nrm_v)
    pltpu.async_copy(tab_hbm.at[src_v], rows_a, sem_ga)

    def pair_body(r, _):
        ba = 2 * r
        bb = 2 * r + 1
        idx_load(bb, src_w, dst_w, nrm_w)
        pltpu.async_copy(tab_hbm.at[src_w], rows_b, sem_gb)
        pltpu.make_async_copy(tab_hbm.at[src_v], rows_a, sem_ga).wait()
        scale(rows_a, nrm_v)
        pltpu.async_copy(rows_a, acc_sh.at[dst_v], sem_sa, add=True)
        pltpu.make_async_copy(tab_hbm.at[src_w], rows_b, sem_gb).wait()
        pltpu.make_async_copy(rows_a, acc_sh.at[dst_v], sem_sa).wait()

        @pl.when(r < NPAIR - 1)
        def _():
            idx_load(ba + 2, src_v, dst_v, nrm_v)

        @pl.when(r < NPAIR - 1)
        def _():
            pltpu.async_copy(tab_hbm.at[src_v], rows_a, sem_ga)
        scale(rows_b, nrm_w)
        pltpu.sync_copy(rows_b, acc_sh.at[dst_w], add=True)
        return 0

    lax.fori_loop(0, NPAIR, pair_body, 0)
    plsc.subcore_barrier()
    band = pl.ds(row0, NBR)
    pltpu.sync_copy(acc_sh.at[band], out_hbm.at[c].at[band])


def _bn(h, gamma, beta, eps=1e-5):
    mu = h.mean(axis=0)
    var = h.var(axis=0)
    return (h - mu) / jnp.sqrt(var + eps) * gamma + beta


def kernel(x, edge_index, batch, edge_attr, W1, b1, g1, be1, W2, b2, g2, be2,
           Wm, bm, gm, bem, Wf1, bf1, gf1, bef1, Wf2, bf2, gf2, bef2, Wf3, bf3):
    src = edge_index[0]
    dst = edge_index[1]

    deg_parts = _deg_kernel(dst, edge_attr)
    deg = deg_parts.sum(axis=0)
    dinv = jnp.where(deg > 0, jax.lax.rsqrt(jnp.maximum(deg, 1e-30)), 0.0)
    nrm = _norm_kernel(src, dst, edge_attr, dinv)

    pad_idx = (jnp.arange(EP - E, dtype=jnp.int32) * 13) % N
    src_p = jnp.concatenate([src, pad_idx])
    dst_p = jnp.concatenate([dst, pad_idx])
    nrm_p = jnp.pad(nrm, (0, EP - E))

    def conv(h_pad):
        hs = []
        t = h_pad
        for _ in range(K):
            parts = _step_kernel(t, src_p, dst_p, nrm_p)
            t = parts[0] + parts[1]
            hs.append(t[:N])
        return hs

    x_pad = jnp.pad(x, ((0, NP - N), (0, 0)))
    h1, h2, h3 = conv(x_pad)
    hcat = jnp.concatenate([x, h1, h2, h3], axis=1)
    x1 = _bn(jax.nn.relu(hcat @ W1 + b1), g1, be1)

    h1, h2, h3 = conv(jnp.pad(x1, ((0, NP - N), (0, 0))))
    hcat = jnp.concatenate([x1, h1, h2, h3], axis=1)
    x2 = _bn(jax.nn.relu(hcat @ W2 + b2), g2, be2)

    out = jnp.concatenate([x1, x2], axis=1)
    out = _bn(jax.nn.relu(out @ Wm + bm), gm, bem)
    out = jax.ops.segment_max(out, batch, num_segments=G)
    out = _bn(jax.nn.relu(out @ Wf1 + bf1), gf1, bef1)
    out = _bn(jax.nn.relu(out @ Wf2 + bf2), gf2, bef2)
    out = out @ Wf3 + bf3
    return jax.nn.log_softmax(out, axis=-1)
